# Initial kernel scaffold; baseline (speedup 1.0000x reference)
import jax
import jax.numpy as jnp
from jax.experimental import pallas as pl


def kernel(x_0, node_idx, edge_idx, weight):
    raise NotImplementedError("write your pallas kernel here")


# speedup vs baseline: 6.9614x; 6.9614x over previous
import functools

import jax
import jax.numpy as jnp
from jax import lax
from jax.experimental import pallas as pl
from jax.experimental.pallas import tpu as pltpu
from jax.experimental.pallas import tpu_sc as plsc

N_NODES = 10000
N_EDGES = 10000
NNZ = 320000
D_IN = 128
D_OUT = 128
HALF = 64

NS = 16
ROWS = 10112
DUMMY = 10016
ROWS_PER_TILE = ROWS // NS
CHUNK = 128
PER_TILE = 20096
NCHUNK = PER_TILE // CHUNK
NNZ_PAD = PER_TILE * NS


def _mm_body(x_ref, w_ref, oa_ref, ob_ref):
    y = jnp.dot(x_ref[...], w_ref[...], preferred_element_type=jnp.float32)
    oa_ref[...] = y[:, :HALF]
    ob_ref[...] = y[:, HALF:]


def _matmul_halves(x0p, weight):
    rb = ROWS // 4
    return pl.pallas_call(
        _mm_body,
        grid=(4,),
        in_specs=[
            pl.BlockSpec((rb, D_IN), lambda i: (i, 0)),
            pl.BlockSpec((D_IN, D_OUT), lambda i: (0, 0)),
        ],
        out_specs=[
            pl.BlockSpec((rb, HALF), lambda i: (i, 0)),
            pl.BlockSpec((rb, HALF), lambda i: (i, 0)),
        ],
        out_shape=[
            jax.ShapeDtypeStruct((ROWS, HALF), jnp.float32),
            jax.ShapeDtypeStruct((ROWS, HALF), jnp.float32),
        ],
    )(x0p, weight)


def _sc_body(xwa, xwb, node_hbm, edge_hbm, zero_hbm, out_a, out_b,
             node_v, edge_v, buf, acc_m, acc_o, sem):
    c = lax.axis_index("c")
    s = lax.axis_index("s")
    r0 = s * ROWS_PER_TILE

    pltpu.sync_copy(node_hbm.at[s], node_v)
    pltpu.sync_copy(edge_hbm.at[s], edge_v)
    pltpu.sync_copy(zero_hbm.at[pl.ds(r0, ROWS_PER_TILE)],
                    acc_m.at[pl.ds(r0, ROWS_PER_TILE)])
    pltpu.sync_copy(zero_hbm.at[pl.ds(r0, ROWS_PER_TILE)],
                    acc_o.at[pl.ds(r0, ROWS_PER_TILE)])
    plsc.subcore_barrier()

    def hop1(table):
        def body(j, carry):
            pltpu.async_copy(table.at[node_v.at[j]], buf, sem).wait()
            pltpu.sync_copy(buf, acc_m.at[edge_v.at[j]], add=True)
            return carry
        lax.fori_loop(0, NCHUNK, body, 0)

    @pl.when(c == 0)
    def _():
        hop1(xwa)

    @pl.when(c == 1)
    def _():
        hop1(xwb)

    plsc.subcore_barrier()

    def hop2(j, carry):
        pltpu.async_copy(acc_m.at[edge_v.at[j]], buf, sem).wait()
        pltpu.sync_copy(buf, acc_o.at[node_v.at[j]], add=True)
        return carry
    lax.fori_loop(0, NCHUNK, hop2, 0)
    plsc.subcore_barrier()

    @pl.when(c == 0)
    def _():
        pltpu.sync_copy(acc_o.at[pl.ds(r0, ROWS_PER_TILE)],
                        out_a.at[pl.ds(r0, ROWS_PER_TILE)])

    @pl.when(c == 1)
    def _():
        pltpu.sync_copy(acc_o.at[pl.ds(r0, ROWS_PER_TILE)],
                        out_b.at[pl.ds(r0, ROWS_PER_TILE)])


_sc_call = pl.kernel(
    _sc_body,
    out_type=[
        jax.ShapeDtypeStruct((ROWS, HALF), jnp.float32),
        jax.ShapeDtypeStruct((ROWS, HALF), jnp.float32),
    ],
    mesh=plsc.VectorSubcoreMesh(core_axis_name="c", subcore_axis_name="s"),
    scratch_types=[
        pltpu.VMEM((NCHUNK, CHUNK), jnp.int32),
        pltpu.VMEM((NCHUNK, CHUNK), jnp.int32),
        pltpu.VMEM((CHUNK, HALF), jnp.float32),
        pltpu.VMEM_SHARED((ROWS, HALF), jnp.float32),
        pltpu.VMEM_SHARED((ROWS, HALF), jnp.float32),
        pltpu.SemaphoreType.DMA,
    ],
    compiler_params=pltpu.CompilerParams(use_tc_tiling_on_sc=False),
)


@jax.jit
def kernel(x_0, node_idx, edge_idx, weight):
    x0p = jnp.zeros((ROWS, D_IN), jnp.float32).at[:N_NODES].set(x_0)
    pad = jnp.full((NNZ_PAD - NNZ,), DUMMY, jnp.int32)
    node3 = jnp.concatenate([node_idx, pad]).reshape(NS, NCHUNK, CHUNK)
    edge3 = jnp.concatenate([edge_idx, pad]).reshape(NS, NCHUNK, CHUNK)
    zeros = jnp.zeros((ROWS, HALF), jnp.float32)
    xwa, xwb = _matmul_halves(x0p, weight)
    out_a, out_b = _sc_call(xwa, xwb, node3, edge3, zeros)
    return jnp.concatenate([out_a[:N_NODES], out_b[:N_NODES]], axis=1)


# speedup vs baseline: 7.0011x; 1.0057x over previous
import functools

import jax
import jax.numpy as jnp
from jax import lax
from jax.experimental import pallas as pl
from jax.experimental.pallas import tpu as pltpu
from jax.experimental.pallas import tpu_sc as plsc

N_NODES = 10000
N_EDGES = 10000
NNZ = 320000
D_IN = 128
D_OUT = 128
HALF = 64

NS = 16
ROWS = 10112
DUMMY = 10016
ROWS_PER_TILE = ROWS // NS
CHUNK = 64
NBUF = 2
NCHUNK = 320
NGROUP = NCHUNK // NBUF
PER_TILE = NCHUNK * CHUNK
NNZ_PAD = PER_TILE * NS


def _mm_body(x_ref, w_ref, oa_ref, ob_ref):
    y = jnp.dot(x_ref[...], w_ref[...], preferred_element_type=jnp.float32)
    oa_ref[...] = y[:, :HALF]
    ob_ref[...] = y[:, HALF:]


def _matmul_halves(x0p, weight):
    rb = ROWS // 4
    return pl.pallas_call(
        _mm_body,
        grid=(4,),
        in_specs=[
            pl.BlockSpec((rb, D_IN), lambda i: (i, 0)),
            pl.BlockSpec((D_IN, D_OUT), lambda i: (0, 0)),
        ],
        out_specs=[
            pl.BlockSpec((rb, HALF), lambda i: (i, 0)),
            pl.BlockSpec((rb, HALF), lambda i: (i, 0)),
        ],
        out_shape=[
            jax.ShapeDtypeStruct((ROWS, HALF), jnp.float32),
            jax.ShapeDtypeStruct((ROWS, HALF), jnp.float32),
        ],
    )(x0p, weight)


def _hop(table, g_idx, s_idx, acc, bufs, sems, zero_hbm):
    for b in range(NBUF):
        pltpu.async_copy(table.at[g_idx.at[b]], bufs.at[b], sems.at[b])

    def group(g, carry):
        for b in range(NBUF):
            j = g * NBUF + b
            pltpu.make_async_copy(zero_hbm.at[pl.ds(0, CHUNK)], bufs.at[b],
                                  sems.at[b]).wait()
            pltpu.sync_copy(bufs.at[b], acc.at[s_idx.at[j]], add=True)
            jn = jnp.minimum(j + NBUF, NCHUNK - 1)
            pltpu.async_copy(table.at[g_idx.at[jn]], bufs.at[b], sems.at[b])
        return carry

    lax.fori_loop(0, NGROUP, group, 0)
    for b in range(NBUF):
        pltpu.make_async_copy(zero_hbm.at[pl.ds(0, CHUNK)], bufs.at[b],
                              sems.at[b]).wait()


def _sc_body(xwa, xwb, node_hbm, edge_hbm, zero_hbm, out_a, out_b,
             node_v, edge_v, bufs, acc_m, acc_o, sems):
    c = lax.axis_index("c")
    s = lax.axis_index("s")
    r0 = s * ROWS_PER_TILE

    pltpu.sync_copy(node_hbm.at[s], node_v)
    pltpu.sync_copy(edge_hbm.at[s], edge_v)
    pltpu.sync_copy(zero_hbm.at[pl.ds(r0, ROWS_PER_TILE)],
                    acc_m.at[pl.ds(r0, ROWS_PER_TILE)])
    pltpu.sync_copy(zero_hbm.at[pl.ds(r0, ROWS_PER_TILE)],
                    acc_o.at[pl.ds(r0, ROWS_PER_TILE)])
    plsc.subcore_barrier()

    @pl.when(c == 0)
    def _():
        _hop(xwa, node_v, edge_v, acc_m, bufs, sems, zero_hbm)

    @pl.when(c == 1)
    def _():
        _hop(xwb, node_v, edge_v, acc_m, bufs, sems, zero_hbm)

    plsc.subcore_barrier()

    _hop(acc_m, edge_v, node_v, acc_o, bufs, sems, zero_hbm)
    plsc.subcore_barrier()

    @pl.when(c == 0)
    def _():
        pltpu.sync_copy(acc_o.at[pl.ds(r0, ROWS_PER_TILE)],
                        out_a.at[pl.ds(r0, ROWS_PER_TILE)])

    @pl.when(c == 1)
    def _():
        pltpu.sync_copy(acc_o.at[pl.ds(r0, ROWS_PER_TILE)],
                        out_b.at[pl.ds(r0, ROWS_PER_TILE)])


_sc_call = pl.kernel(
    _sc_body,
    out_type=[
        jax.ShapeDtypeStruct((ROWS, HALF), jnp.float32),
        jax.ShapeDtypeStruct((ROWS, HALF), jnp.float32),
    ],
    mesh=plsc.VectorSubcoreMesh(core_axis_name="c", subcore_axis_name="s"),
    scratch_types=[
        pltpu.VMEM((NCHUNK, CHUNK), jnp.int32),
        pltpu.VMEM((NCHUNK, CHUNK), jnp.int32),
        pltpu.VMEM((NBUF, CHUNK, HALF), jnp.float32),
        pltpu.VMEM_SHARED((ROWS, HALF), jnp.float32),
        pltpu.VMEM_SHARED((ROWS, HALF), jnp.float32),
        pltpu.SemaphoreType.DMA((NBUF,)),
    ],
    compiler_params=pltpu.CompilerParams(use_tc_tiling_on_sc=False),
)


@jax.jit
def kernel(x_0, node_idx, edge_idx, weight):
    x0p = jnp.zeros((ROWS, D_IN), jnp.float32).at[:N_NODES].set(x_0)
    pad = jnp.full((NNZ_PAD - NNZ,), DUMMY, jnp.int32)
    node3 = jnp.concatenate([node_idx, pad]).reshape(NS, NCHUNK, CHUNK)
    edge3 = jnp.concatenate([edge_idx, pad]).reshape(NS, NCHUNK, CHUNK)
    zeros = jnp.zeros((ROWS, HALF), jnp.float32)
    xwa, xwb = _matmul_halves(x0p, weight)
    out_a, out_b = _sc_call(xwa, xwb, node3, edge3, zeros)
    return jnp.concatenate([out_a[:N_NODES], out_b[:N_NODES]], axis=1)
